# Initial kernel scaffold; baseline (speedup 1.0000x reference)
import jax
import jax.numpy as jnp
from jax.experimental import pallas as pl


def kernel(y_pred, y_true, index_p, u_pos):
    raise NotImplementedError("write your pallas kernel here")


# speedup vs baseline: 1.3489x; 1.3489x over previous
import functools

import jax
import jax.numpy as jnp
from jax import lax
from jax.experimental import pallas as pl
from jax.experimental.pallas import tpu as pltpu
from jax.experimental.pallas import tpu_sc as plsc

_N_POS = 2048
_N_NEG = 14336
_POS_LEN = 100000
_MARGIN = 1.0
_BETA = 0.2
_SCALE = 1.0 / (_N_POS * _N_NEG * _BETA)

_NC = 2
_NS = 16
_NW = _NC * _NS
_B_PER_W = _N_POS // _NW

@functools.lru_cache(maxsize=1)
def _gather_u_kernel():
    mesh = plsc.VectorSubcoreMesh(core_axis_name="c", subcore_axis_name="s")

    @functools.partial(
        pl.kernel,
        mesh=mesh,
        out_type=jax.ShapeDtypeStruct((_N_POS,), jnp.float32),
        scratch_types=[
            pltpu.VMEM((_B_PER_W,), jnp.int32),
            pltpu.VMEM((_B_PER_W,), jnp.float32),
            pltpu.SemaphoreType.DMA,
        ],
    )
    def _gather_u(idx_hbm, u_hbm, out_hbm, idx_v, rows_v, sem):
        wid = lax.axis_index("s") * _NC + lax.axis_index("c")
        base = wid * _B_PER_W
        pltpu.sync_copy(idx_hbm.at[pl.ds(base, _B_PER_W)], idx_v)
        pltpu.async_copy(u_hbm.at[idx_v], rows_v, sem).wait()
        pltpu.sync_copy(rows_v, out_hbm.at[pl.ds(base, _B_PER_W)])

    return _gather_u


_BLK_R = 256
_BLK_C = 3584


def _dense_body(fp_ref, fn_ref, u_ref, out_ref):
    i = pl.program_id(0)
    j = pl.program_id(1)

    @pl.when((i == 0) & (j == 0))
    def _init():
        out_ref[0, 0] = 0.0

    a = _MARGIN - fp_ref[...]
    c = jnp.sqrt(jnp.maximum(u_ref[...], 0.0)) - a
    x = fn_ref[...]
    d = a + x
    s = jnp.where(x > c, d * d, 0.0)
    out_ref[0, 0] += jnp.sum(s)

    @pl.when((i == pl.num_programs(0) - 1) & (j == pl.num_programs(1) - 1))
    def _finish():
        out_ref[0, 0] = out_ref[0, 0] * _SCALE


def _dense(f_ps, f_ns, u_sel):
    grid = (_N_POS // _BLK_R, _N_NEG // _BLK_C)
    return pl.pallas_call(
        _dense_body,
        grid=grid,
        in_specs=[
            pl.BlockSpec((_BLK_R, 1), lambda i, j: (i, 0)),
            pl.BlockSpec((1, _BLK_C), lambda i, j: (0, j)),
            pl.BlockSpec((_BLK_R, 1), lambda i, j: (i, 0)),
        ],
        out_specs=pl.BlockSpec(
            (1, 1), lambda i, j: (0, 0), memory_space=pltpu.SMEM
        ),
        out_shape=jax.ShapeDtypeStruct((1, 1), jnp.float32),
        compiler_params=pltpu.CompilerParams(
            dimension_semantics=("arbitrary", "arbitrary"),
        ),
    )(f_ps, f_ns, u_sel)


def kernel(y_pred, y_true, index_p, u_pos):
    del y_true
    yp = y_pred.reshape(-1)
    f_ps = yp[:_N_POS].reshape(_N_POS, 1)
    f_ns = yp[_N_POS:].reshape(1, _N_NEG)
    idx = index_p[:_N_POS]
    u_sel = _gather_u_kernel()(idx, u_pos.reshape(-1)).reshape(_N_POS, 1)
    out = _dense(f_ps, f_ns, u_sel)
    return out[0, 0]


# speedup vs baseline: 1.4309x; 1.0607x over previous
import functools

import jax
import jax.numpy as jnp
from jax import lax
from jax.experimental import pallas as pl
from jax.experimental.pallas import tpu as pltpu
from jax.experimental.pallas import tpu_sc as plsc

_N_POS = 2048
_N_NEG = 14336
_POS_LEN = 100000
_MARGIN = 1.0
_BETA = 0.2
_SCALE = 1.0 / (_N_POS * _N_NEG * _BETA)

_NC = 2
_NS = 16
_NW = _NC * _NS
_B_PER_W = _N_POS // _NW

@functools.lru_cache(maxsize=1)
def _gather_u_kernel():
    mesh = plsc.VectorSubcoreMesh(core_axis_name="c", subcore_axis_name="s")

    @functools.partial(
        pl.kernel,
        mesh=mesh,
        out_type=jax.ShapeDtypeStruct((_N_POS,), jnp.float32),
        scratch_types=[
            pltpu.VMEM((_B_PER_W,), jnp.int32),
            pltpu.VMEM((_B_PER_W,), jnp.float32),
            pltpu.SemaphoreType.DMA,
        ],
    )
    def _gather_u(idx_hbm, u_hbm, out_hbm, idx_v, rows_v, sem):
        wid = lax.axis_index("s") * _NC + lax.axis_index("c")
        base = wid * _B_PER_W
        pltpu.sync_copy(idx_hbm.at[pl.ds(base, _B_PER_W)], idx_v)
        pltpu.async_copy(u_hbm.at[idx_v], rows_v, sem).wait()
        pltpu.sync_copy(rows_v, out_hbm.at[pl.ds(base, _B_PER_W)])

    return _gather_u


_BLK_R = 256


def _dense_body(fp_ref, fn_ref, fnc_ref, u_ref, out_ref):
    i = pl.program_id(0)

    @pl.when(i == 0)
    def _init():
        out_ref[0, 0] = 0.0

    a = _MARGIN - fp_ref[...]
    c = jnp.sqrt(jnp.maximum(u_ref[...], 0.0)) - a
    x = fn_ref[...]
    mf = jnp.where(x > c, 1.0, 0.0)
    xc = fnc_ref[...]
    basis = jnp.concatenate(
        [jnp.ones_like(xc), xc, xc * xc], axis=1)
    st = jax.lax.dot_general(
        mf, basis, (((1,), (0,)), ((), ())),
        preferred_element_type=jnp.float32)
    row = (a * a) * st[:, 0:1] + (2.0 * a) * st[:, 1:2] + st[:, 2:3]
    out_ref[0, 0] += jnp.sum(row)

    @pl.when(i == pl.num_programs(0) - 1)
    def _finish():
        out_ref[0, 0] = out_ref[0, 0] * _SCALE


def _dense(f_ps, f_ns, f_ns_col, u_sel):
    grid = (_N_POS // _BLK_R,)
    return pl.pallas_call(
        _dense_body,
        grid=grid,
        in_specs=[
            pl.BlockSpec((_BLK_R, 1), lambda i: (i, 0)),
            pl.BlockSpec((1, _N_NEG), lambda i: (0, 0)),
            pl.BlockSpec((_N_NEG, 1), lambda i: (0, 0)),
            pl.BlockSpec((_BLK_R, 1), lambda i: (i, 0)),
        ],
        out_specs=pl.BlockSpec(
            (1, 1), lambda i: (0, 0), memory_space=pltpu.SMEM
        ),
        out_shape=jax.ShapeDtypeStruct((1, 1), jnp.float32),
        compiler_params=pltpu.CompilerParams(
            dimension_semantics=("arbitrary",),
        ),
    )(f_ps, f_ns, f_ns_col, u_sel)


def kernel(y_pred, y_true, index_p, u_pos):
    del y_true
    yp = y_pred.reshape(-1)
    f_ps = yp[:_N_POS].reshape(_N_POS, 1)
    f_ns = yp[_N_POS:].reshape(1, _N_NEG)
    f_ns_col = yp[_N_POS:].reshape(_N_NEG, 1)
    idx = index_p[:_N_POS]
    u_sel = _gather_u_kernel()(idx, u_pos.reshape(-1)).reshape(_N_POS, 1)
    out = _dense(f_ps, f_ns, f_ns_col, u_sel)
    return out[0, 0]


# speedup vs baseline: 1.9203x; 1.3420x over previous
import functools

import jax
import jax.numpy as jnp
from jax import lax
from jax.experimental import pallas as pl
from jax.experimental.pallas import tpu as pltpu
from jax.experimental.pallas import tpu_sc as plsc

_N_POS = 2048
_N_NEG = 14336
_POS_LEN = 100000
_MARGIN = 1.0
_BETA = 0.2
_SCALE = 1.0 / (_N_POS * _N_NEG * _BETA)

_NC = 2
_NS = 16
_NW = _NC * _NS
_B_PER_W = _N_POS // _NW

@functools.lru_cache(maxsize=1)
def _gather_u_kernel():
    mesh = plsc.VectorSubcoreMesh(core_axis_name="c", subcore_axis_name="s")

    @functools.partial(
        pl.kernel,
        mesh=mesh,
        out_type=jax.ShapeDtypeStruct((_N_POS,), jnp.float32),
        scratch_types=[
            pltpu.VMEM((_B_PER_W,), jnp.int32),
            pltpu.VMEM((_B_PER_W,), jnp.float32),
            pltpu.SemaphoreType.DMA,
        ],
    )
    def _gather_u(idx_hbm, u_hbm, out_hbm, idx_v, rows_v, sem):
        wid = lax.axis_index("s") * _NC + lax.axis_index("c")
        base = wid * _B_PER_W
        pltpu.sync_copy(idx_hbm.at[pl.ds(base, _B_PER_W)], idx_v)
        pltpu.async_copy(u_hbm.at[idx_v], rows_v, sem).wait()
        pltpu.sync_copy(rows_v, out_hbm.at[pl.ds(base, _B_PER_W)])

    return _gather_u


_BLK_R = 256


def _dense_body(fp_ref, fn_ref, u_ref, out_ref):
    i = pl.program_id(0)

    @pl.when(i == 0)
    def _init():
        out_ref[0, 0] = 0.0

    a = _MARGIN - fp_ref[...]
    c = jnp.sqrt(jnp.maximum(u_ref[...], 0.0)) - a
    x = fn_ref[...]
    mf = jnp.where(x > c, 1.0, 0.0)
    basis_t = jnp.concatenate(
        [jnp.ones_like(x), x, x * x], axis=0)
    st = jax.lax.dot_general(
        mf, basis_t, (((1,), (1,)), ((), ())),
        preferred_element_type=jnp.float32)
    row = (a * a) * st[:, 0:1] + (2.0 * a) * st[:, 1:2] + st[:, 2:3]
    out_ref[0, 0] += jnp.sum(row)

    @pl.when(i == pl.num_programs(0) - 1)
    def _finish():
        out_ref[0, 0] = out_ref[0, 0] * _SCALE


def _dense(f_ps, f_ns, u_sel):
    grid = (_N_POS // _BLK_R,)
    return pl.pallas_call(
        _dense_body,
        grid=grid,
        in_specs=[
            pl.BlockSpec((_BLK_R, 1), lambda i: (i, 0)),
            pl.BlockSpec((1, _N_NEG), lambda i: (0, 0)),
            pl.BlockSpec((_BLK_R, 1), lambda i: (i, 0)),
        ],
        out_specs=pl.BlockSpec(
            (1, 1), lambda i: (0, 0), memory_space=pltpu.SMEM
        ),
        out_shape=jax.ShapeDtypeStruct((1, 1), jnp.float32),
        compiler_params=pltpu.CompilerParams(
            dimension_semantics=("arbitrary",),
        ),
    )(f_ps, f_ns, u_sel)


def kernel(y_pred, y_true, index_p, u_pos):
    del y_true
    yp = y_pred.reshape(-1)
    f_ps = yp[:_N_POS].reshape(_N_POS, 1)
    f_ns = yp[_N_POS:].reshape(1, _N_NEG)
    idx = index_p[:_N_POS]
    u_sel = _gather_u_kernel()(idx, u_pos.reshape(-1)).reshape(_N_POS, 1)
    out = _dense(f_ps, f_ns, u_sel)
    return out[0, 0]


# speedup vs baseline: 2.4520x; 1.2769x over previous
import functools

import jax
import jax.numpy as jnp
from jax import lax
from jax.experimental import pallas as pl
from jax.experimental.pallas import tpu as pltpu
from jax.experimental.pallas import tpu_sc as plsc

_N_POS = 2048
_N_NEG = 14336
_POS_LEN = 100000
_MARGIN = 1.0
_BETA = 0.2
_SCALE = 1.0 / (_N_POS * _N_NEG * _BETA)

_NC = 2
_NS = 16
_NW = _NC * _NS
_B_PER_W = _N_POS // _NW

@functools.lru_cache(maxsize=1)
def _gather_u_kernel():
    mesh = plsc.VectorSubcoreMesh(core_axis_name="c", subcore_axis_name="s")

    @functools.partial(
        pl.kernel,
        mesh=mesh,
        out_type=jax.ShapeDtypeStruct((_N_POS,), jnp.float32),
        scratch_types=[
            pltpu.VMEM((_B_PER_W,), jnp.int32),
            pltpu.VMEM((_B_PER_W,), jnp.float32),
            pltpu.SemaphoreType.DMA,
        ],
    )
    def _gather_u(idx_hbm, u_hbm, out_hbm, idx_v, rows_v, sem):
        wid = lax.axis_index("s") * _NC + lax.axis_index("c")
        base = wid * _B_PER_W
        pltpu.sync_copy(idx_hbm.at[pl.ds(base, _B_PER_W)], idx_v)
        pltpu.async_copy(u_hbm.at[idx_v], rows_v, sem).wait()
        pltpu.sync_copy(rows_v, out_hbm.at[pl.ds(base, _B_PER_W)])

    return _gather_u


def _dense_body(fp_col_ref, fp_row_ref, fn_ref, u_ref, out_ref):
    a_col = _MARGIN - fp_col_ref[...]
    c = jnp.sqrt(jnp.maximum(u_ref[...], 0.0)) - a_col
    x = fn_ref[...]
    mf = jnp.where(x > c, 1.0, 0.0)
    a_row = _MARGIN - fp_row_ref[...]
    w = jnp.concatenate(
        [a_row * a_row, 2.0 * a_row, jnp.ones_like(a_row)], axis=0
    )
    cstat = jax.lax.dot_general(
        w, mf, (((1,), (0,)), ((), ())),
        preferred_element_type=jnp.float32)
    tot = cstat[0:1, :] + cstat[1:2, :] * x + cstat[2:3, :] * (x * x)
    out_ref[0, 0] = jnp.sum(tot) * _SCALE


def _dense(f_ps_col, f_ps_row, f_ns, u_sel):
    return pl.pallas_call(
        _dense_body,
        in_specs=[
            pl.BlockSpec((_N_POS, 1), lambda: (0, 0)),
            pl.BlockSpec((1, _N_POS), lambda: (0, 0)),
            pl.BlockSpec((1, _N_NEG), lambda: (0, 0)),
            pl.BlockSpec((_N_POS, 1), lambda: (0, 0)),
        ],
        out_specs=pl.BlockSpec(
            (1, 1), lambda: (0, 0), memory_space=pltpu.SMEM
        ),
        out_shape=jax.ShapeDtypeStruct((1, 1), jnp.float32),
    )(f_ps_col, f_ps_row, f_ns, u_sel)


def kernel(y_pred, y_true, index_p, u_pos):
    del y_true
    yp = y_pred.reshape(-1)
    f_ps = yp[:_N_POS].reshape(_N_POS, 1)
    f_ps_row = yp[:_N_POS].reshape(1, _N_POS)
    f_ns = yp[_N_POS:].reshape(1, _N_NEG)
    idx = index_p[:_N_POS]
    u_sel = _gather_u_kernel()(idx, u_pos.reshape(-1)).reshape(_N_POS, 1)
    out = _dense(f_ps, f_ps_row, f_ns, u_sel)
    return out[0, 0]


# speedup vs baseline: 2.5681x; 1.0473x over previous
import functools

import jax
import jax.numpy as jnp
from jax import lax
from jax.experimental import pallas as pl
from jax.experimental.pallas import tpu as pltpu
from jax.experimental.pallas import tpu_sc as plsc

_N_POS = 2048
_N_NEG = 14336
_POS_LEN = 100000
_MARGIN = 1.0
_BETA = 0.2
_SCALE = 1.0 / (_N_POS * _N_NEG * _BETA)

_NC = 2
_NS = 16
_NW = _NC * _NS
_B_PER_W = _N_POS // _NW

@functools.lru_cache(maxsize=2)
def _gather_u_kernel(ncores=1):
    mesh = plsc.VectorSubcoreMesh(
        core_axis_name="c", subcore_axis_name="s", num_cores=ncores
    )
    b_per_w = _N_POS // (_NS * ncores)

    @functools.partial(
        pl.kernel,
        mesh=mesh,
        out_type=jax.ShapeDtypeStruct((_N_POS,), jnp.float32),
        scratch_types=[
            pltpu.VMEM((b_per_w,), jnp.int32),
            pltpu.VMEM((b_per_w,), jnp.float32),
            pltpu.SemaphoreType.DMA,
        ],
    )
    def _gather_u(idx_hbm, u_hbm, out_hbm, idx_v, rows_v, sem):
        wid = lax.axis_index("s") * ncores + lax.axis_index("c")
        base = wid * b_per_w
        pltpu.sync_copy(idx_hbm.at[pl.ds(base, b_per_w)], idx_v)
        pltpu.async_copy(u_hbm.at[idx_v], rows_v, sem).wait()
        pltpu.sync_copy(rows_v, out_hbm.at[pl.ds(base, b_per_w)])

    return _gather_u


def _dense_body(fp_col_ref, fp_row_ref, fn_ref, u_ref, out_ref):
    a_col = _MARGIN - fp_col_ref[...]
    c = jnp.sqrt(jnp.maximum(u_ref[...], 0.0)) - a_col
    x = fn_ref[...]
    mf = jnp.where(x > c, 1.0, 0.0)
    a_row = _MARGIN - fp_row_ref[...]
    w = jnp.concatenate(
        [a_row * a_row, 2.0 * a_row, jnp.ones_like(a_row)], axis=0
    )
    cstat = jax.lax.dot_general(
        w, mf, (((1,), (0,)), ((), ())),
        preferred_element_type=jnp.float32)
    tot = cstat[0:1, :] + cstat[1:2, :] * x + cstat[2:3, :] * (x * x)
    out_ref[0, 0] = jnp.sum(tot) * _SCALE


def _dense(f_ps_col, f_ps_row, f_ns, u_sel):
    return pl.pallas_call(
        _dense_body,
        in_specs=[
            pl.BlockSpec((_N_POS, 1), lambda: (0, 0)),
            pl.BlockSpec((1, _N_POS), lambda: (0, 0)),
            pl.BlockSpec((1, _N_NEG), lambda: (0, 0)),
            pl.BlockSpec((_N_POS, 1), lambda: (0, 0)),
        ],
        out_specs=pl.BlockSpec(
            (1, 1), lambda: (0, 0), memory_space=pltpu.SMEM
        ),
        out_shape=jax.ShapeDtypeStruct((1, 1), jnp.float32),
    )(f_ps_col, f_ps_row, f_ns, u_sel)


def kernel(y_pred, y_true, index_p, u_pos):
    del y_true
    f_ps = y_pred[:_N_POS]
    f_ps_row = f_ps.reshape(1, _N_POS)
    f_ns = y_pred[_N_POS:].reshape(1, _N_NEG)
    idx = index_p[:_N_POS]
    u_sel = _gather_u_kernel()(idx, u_pos.reshape(-1)).reshape(_N_POS, 1)
    out = _dense(f_ps, f_ps_row, f_ns, u_sel)
    return out[0, 0]
